# baseline (speedup 1.0000x reference)
import jax
import jax.numpy as jnp
from jax.experimental import pallas as pl
from jax.experimental.pallas import tpu as pltpu
from jax.experimental.pallas import tpu_sc as plsc


_NT = 16
_NSC = 2
_CH = 128
_G = 8
_NBUF = 2


def _sc_scatter_sum(x2d, srcs, dsts, zrows, B, N, D, ngroups, rows_pt, n_pad):
    mesh = plsc.VectorSubcoreMesh(core_axis_name="c", subcore_axis_name="s")
    b_per_sc = B // _NSC
    wr_last = N - (_NT - 1) * rows_pt
    zr_last = n_pad - (_NT - 1) * rows_pt
    assert wr_last > 0 and wr_last % 8 == 0 and rows_pt % 8 == 0
    assert 0 < zr_last <= rows_pt and zr_last % 8 == 0

    def body(x_hbm, src_hbm, dst_hbm, z_hbm, out_hbm, src_v, dst_v,
             rows0, rows1, gsem0, gsem1, ssem0, ssem1, acc_sh):
        cid = jax.lax.axis_index("c")
        sid = jax.lax.axis_index("s")
        rows = (rows0, rows1)
        gsems = (gsem0, gsem1)
        ssems = (ssem0, ssem1)
        for b_loc in range(b_per_sc):
            b = b_loc * _NSC + cid

            @pl.when(sid < _NT - 1)
            def _():
                pltpu.sync_copy(z_hbm, acc_sh.at[pl.ds(sid * rows_pt, rows_pt)])

            @pl.when(sid == _NT - 1)
            def _():
                pltpu.sync_copy(
                    z_hbm.at[pl.ds(0, zr_last)],
                    acc_sh.at[pl.ds((_NT - 1) * rows_pt, zr_last)])

            plsc.subcore_barrier()

            @pl.loop(0, ngroups)
            def _(g):
                pltpu.sync_copy(src_hbm.at[(b * _NT + sid) * ngroups + g], src_v)
                pltpu.sync_copy(dst_hbm.at[sid * ngroups + g], dst_v)

                def gather(j):
                    k = j % _NBUF
                    return pltpu.make_async_copy(
                        x_hbm.at[src_v.at[j]], rows[k], gsems[k])

                gather(0).start()
                gather(1).start()
                for j in range(_G):
                    k = j % _NBUF
                    gather(j).wait()
                    pltpu.sync_copy(rows[k], acc_sh.at[dst_v.at[j]], add=True)
                    if j + 2 < _G:
                        gather(j + 2).start()

            plsc.subcore_barrier()

            @pl.when(sid < _NT - 1)
            def _():
                pltpu.sync_copy(
                    acc_sh.at[pl.ds(sid * rows_pt, rows_pt)],
                    out_hbm.at[pl.ds(b * N + sid * rows_pt, rows_pt)],
                )

            @pl.when(sid == _NT - 1)
            def _():
                pltpu.sync_copy(
                    acc_sh.at[pl.ds((_NT - 1) * rows_pt, wr_last)],
                    out_hbm.at[pl.ds(b * N + (_NT - 1) * rows_pt, wr_last)],
                )

            plsc.subcore_barrier()

    kern = pl.kernel(
        body,
        out_type=jax.ShapeDtypeStruct((B * N, D), jnp.float32),
        mesh=mesh,
        scratch_types=[
            pltpu.VMEM((_G, _CH), jnp.int32),
            pltpu.VMEM((_G, _CH), jnp.int32),
            pltpu.VMEM((_CH, D), jnp.float32),
            pltpu.VMEM((_CH, D), jnp.float32),
            pltpu.SemaphoreType.DMA,
            pltpu.SemaphoreType.DMA,
            pltpu.SemaphoreType.DMA,
            pltpu.SemaphoreType.DMA,
            pltpu.VMEM_SHARED((n_pad, D), jnp.float32),
        ],
    )
    return kern(x2d, srcs, dsts, zrows)


def _self_body(x_ref, ws_ref, b_ref, o_ref):
    h = jnp.dot(x_ref[...], ws_ref[...], preferred_element_type=jnp.float32)
    o_ref[...] = h + b_ref[...]


def _self_matmul(x2d, wsT, bias):
    M, D = x2d.shape
    BM = 2000
    return pl.pallas_call(
        _self_body,
        grid=(M // BM,),
        in_specs=[
            pl.BlockSpec((BM, D), lambda i: (i, 0)),
            pl.BlockSpec((D, D), lambda i: (0, 0)),
            pl.BlockSpec((1, D), lambda i: (0, 0)),
        ],
        out_specs=pl.BlockSpec((BM, D), lambda i: (i, 0)),
        out_shape=jax.ShapeDtypeStruct((M, D), jnp.float32),
    )(x2d, wsT, bias)


def _combine_body(h_ref, agg_ref, wn_ref, o_ref):
    h = h_ref[...] + jnp.dot(agg_ref[...], wn_ref[...],
                             preferred_element_type=jnp.float32)
    o_ref[...] = jnp.maximum(h, 0.0)


def _combine(h_pre, agg2d, wnT):
    M, D = h_pre.shape
    BM = 2000
    return pl.pallas_call(
        _combine_body,
        grid=(M // BM,),
        in_specs=[
            pl.BlockSpec((BM, D), lambda i: (i, 0)),
            pl.BlockSpec((BM, D), lambda i: (i, 0)),
            pl.BlockSpec((D, D), lambda i: (0, 0)),
        ],
        out_specs=pl.BlockSpec((BM, D), lambda i: (i, 0)),
        out_shape=jax.ShapeDtypeStruct((M, D), jnp.float32),
    )(h_pre, agg2d, wnT)


def kernel(x, edge_index, W_self, b_self, W_neigh, b_neigh):
    B, N, D = x.shape
    E = edge_index.shape[0]

    chunk_all = _NT * _G * _CH
    e_pad = ((E + chunk_all - 1) // chunk_all) * chunk_all
    ngroups = e_pad // chunk_all
    src = edge_index[:, 0].astype(jnp.int32)
    dst = edge_index[:, 1].astype(jnp.int32)
    pad = e_pad - E
    if pad:
        src = jnp.concatenate([src, jnp.zeros((pad,), jnp.int32)])
        dst = jnp.concatenate([dst, jnp.full((pad,), N, jnp.int32)])

    n_pad = ((N + 1 + 7) // 8) * 8
    rows_pt = (((n_pad + _NT - 1) // _NT + 7) // 8) * 8

    boffs = (jnp.arange(B, dtype=jnp.int32) * N)[:, None]
    srcs = (src[None, :] + boffs).reshape(B * _NT * ngroups, _G, _CH)
    dsts = dst.reshape(_NT * ngroups, _G, _CH)
    zrows = jnp.zeros((rows_pt, D), jnp.float32)

    x2d = x.reshape(B * N, D)
    agg2d = _sc_scatter_sum(x2d, srcs, dsts, zrows, B, N, D, ngroups, rows_pt, n_pad)

    bias = (b_self + b_neigh).reshape(1, D)
    h_pre = _self_matmul(x2d, W_self.T, bias)
    out2d = _combine(h_pre, agg2d, W_neigh.T)
    return out2d.reshape(B, N, D)


# speedup vs baseline: 1.0528x; 1.0528x over previous
import jax
import jax.numpy as jnp
from jax.experimental import pallas as pl
from jax.experimental.pallas import tpu as pltpu
from jax.experimental.pallas import tpu_sc as plsc


_NT = 16
_NSC = 2
_CH = 128
_G = 8
_NBUF = 2


def _sc_scatter_sum(x2d, srcs, dsts, zrows, B, N, D, ngroups, rows_pt, n_pad):
    mesh = plsc.VectorSubcoreMesh(core_axis_name="c", subcore_axis_name="s")
    b_per_sc = B // _NSC
    wr_last = N - (_NT - 1) * rows_pt
    zr_last = n_pad - (_NT - 1) * rows_pt
    assert wr_last > 0 and wr_last % 8 == 0 and rows_pt % 8 == 0
    assert 0 < zr_last <= rows_pt and zr_last % 8 == 0

    assert ngroups % 2 == 0

    def body(x_hbm, src_hbm, dst_hbm, z_hbm, out_hbm,
             src_va, dst_va, src_vb, dst_vb,
             rows0, rows1, gsem0, gsem1, isema, isemb, acc_sh):
        cid = jax.lax.axis_index("c")
        sid = jax.lax.axis_index("s")
        rows = (rows0, rows1)
        gsems = (gsem0, gsem1)
        idx = ((src_va, dst_va, isema), (src_vb, dst_vb, isemb))

        def idx_copies(b, g, p):
            sv, dv, sem = idx[p]
            return (pltpu.make_async_copy(
                        src_hbm.at[(b * _NT + sid) * ngroups + g], sv, sem),
                    pltpu.make_async_copy(
                        dst_hbm.at[sid * ngroups + g], dv, sem))
        for b_loc in range(b_per_sc):
            b = b_loc * _NSC + cid

            @pl.when(sid < _NT - 1)
            def _():
                pltpu.sync_copy(z_hbm, acc_sh.at[pl.ds(sid * rows_pt, rows_pt)])

            @pl.when(sid == _NT - 1)
            def _():
                pltpu.sync_copy(
                    z_hbm.at[pl.ds(0, zr_last)],
                    acc_sh.at[pl.ds((_NT - 1) * rows_pt, zr_last)])

            plsc.subcore_barrier()

            for c in idx_copies(b, 0, 0):
                c.start()

            @pl.loop(0, ngroups, step=2)
            def _(g):
                for c in idx_copies(b, g, 0):
                    c.wait()
                for c in idx_copies(b, g + 1, 1):
                    c.start()

                def gather(jj):
                    sv = idx[jj // _G][0]
                    return pltpu.make_async_copy(
                        x_hbm.at[sv.at[jj % _G]], rows[jj % _NBUF],
                        gsems[jj % _NBUF])

                gather(0).start()
                gather(1).start()
                for jj in range(2 * _G):
                    if jj == _G - 2:
                        for c in idx_copies(b, g + 1, 1):
                            c.wait()
                    if jj == _G:
                        @pl.when(g + 2 < ngroups)
                        def _():
                            for c in idx_copies(b, g + 2, 0):
                                c.start()
                    gather(jj).wait()
                    dv = idx[jj // _G][1]
                    pltpu.sync_copy(rows[jj % _NBUF],
                                    acc_sh.at[dv.at[jj % _G]], add=True)
                    if jj + 2 < 2 * _G:
                        gather(jj + 2).start()

            plsc.subcore_barrier()

            @pl.when(sid < _NT - 1)
            def _():
                pltpu.sync_copy(
                    acc_sh.at[pl.ds(sid * rows_pt, rows_pt)],
                    out_hbm.at[pl.ds(b * N + sid * rows_pt, rows_pt)],
                )

            @pl.when(sid == _NT - 1)
            def _():
                pltpu.sync_copy(
                    acc_sh.at[pl.ds((_NT - 1) * rows_pt, wr_last)],
                    out_hbm.at[pl.ds(b * N + (_NT - 1) * rows_pt, wr_last)],
                )

            plsc.subcore_barrier()

    kern = pl.kernel(
        body,
        out_type=jax.ShapeDtypeStruct((B * N, D), jnp.float32),
        mesh=mesh,
        scratch_types=[
            pltpu.VMEM((_G, _CH), jnp.int32),
            pltpu.VMEM((_G, _CH), jnp.int32),
            pltpu.VMEM((_G, _CH), jnp.int32),
            pltpu.VMEM((_G, _CH), jnp.int32),
            pltpu.VMEM((_CH, D), jnp.float32),
            pltpu.VMEM((_CH, D), jnp.float32),
            pltpu.SemaphoreType.DMA,
            pltpu.SemaphoreType.DMA,
            pltpu.SemaphoreType.DMA,
            pltpu.SemaphoreType.DMA,
            pltpu.VMEM_SHARED((n_pad, D), jnp.float32),
        ],
    )
    return kern(x2d, srcs, dsts, zrows)


def _combine_body(x_ref, agg_ref, ws_ref, wn_ref, b_ref, o_ref):
    h = jnp.dot(x_ref[...], ws_ref[...], preferred_element_type=jnp.float32)
    h = h + jnp.dot(agg_ref[...], wn_ref[...], preferred_element_type=jnp.float32)
    o_ref[...] = jnp.maximum(h + b_ref[...], 0.0)


def _combine(x2d, agg2d, wsT, wnT, bias):
    M, D = x2d.shape
    BM = 2000
    grid = (M // BM,)
    return pl.pallas_call(
        _combine_body,
        grid=grid,
        in_specs=[
            pl.BlockSpec((BM, D), lambda i: (i, 0)),
            pl.BlockSpec((BM, D), lambda i: (i, 0)),
            pl.BlockSpec((D, D), lambda i: (0, 0)),
            pl.BlockSpec((D, D), lambda i: (0, 0)),
            pl.BlockSpec((1, D), lambda i: (0, 0)),
        ],
        out_specs=pl.BlockSpec((BM, D), lambda i: (i, 0)),
        out_shape=jax.ShapeDtypeStruct((M, D), jnp.float32),
    )(x2d, agg2d, wsT, wnT, bias)


def kernel(x, edge_index, W_self, b_self, W_neigh, b_neigh):
    B, N, D = x.shape
    E = edge_index.shape[0]

    chunk_all = _NT * _G * _CH
    e_pad = ((E + chunk_all - 1) // chunk_all) * chunk_all
    ngroups = e_pad // chunk_all
    src = edge_index[:, 0].astype(jnp.int32)
    dst = edge_index[:, 1].astype(jnp.int32)
    pad = e_pad - E
    if pad:
        src = jnp.concatenate([src, jnp.zeros((pad,), jnp.int32)])
        dst = jnp.concatenate([dst, jnp.full((pad,), N, jnp.int32)])

    n_pad = ((N + 1 + 7) // 8) * 8
    rows_pt = (((n_pad + _NT - 1) // _NT + 7) // 8) * 8

    boffs = (jnp.arange(B, dtype=jnp.int32) * N)[:, None]
    srcs = (src[None, :] + boffs).reshape(B * _NT * ngroups, _G, _CH)
    dsts = dst.reshape(_NT * ngroups, _G, _CH)
    zrows = jnp.zeros((rows_pt, D), jnp.float32)

    x2d = x.reshape(B * N, D)
    agg2d = _sc_scatter_sum(x2d, srcs, dsts, zrows, B, N, D, ngroups, rows_pt, n_pad)

    bias = (b_self + b_neigh).reshape(1, D)
    out2d = _combine(x2d, agg2d, W_self.T, W_neigh.T, bias)
    return out2d.reshape(B, N, D)


# speedup vs baseline: 1.0739x; 1.0200x over previous
import jax
import jax.numpy as jnp
from jax.experimental import pallas as pl
from jax.experimental.pallas import tpu as pltpu
from jax.experimental.pallas import tpu_sc as plsc


_NT = 16
_NSC = 2
_CH = 128
_G = 8
_NBUF = 2


def _sc_scatter_sum(x2d, srcs, dsts, zrows, B, N, D, ngroups, rows_pt, n_pad):
    mesh = plsc.VectorSubcoreMesh(core_axis_name="c", subcore_axis_name="s")
    b_per_sc = B // _NSC
    wr_last = N - (_NT - 1) * rows_pt
    zr_last = n_pad - (_NT - 1) * rows_pt
    assert wr_last > 0 and wr_last % 8 == 0 and rows_pt % 8 == 0
    assert 0 < zr_last <= rows_pt and zr_last % 8 == 0

    assert ngroups % 2 == 0

    def body(x_hbm, src_hbm, dst_hbm, z_hbm, out_hbm,
             src_va, dst_va, src_vb, dst_vb,
             rows0, rows1, gsem0, gsem1, isema, isemb, acc_sh):
        cid = jax.lax.axis_index("c")
        sid = jax.lax.axis_index("s")
        rows = (rows0, rows1)
        gsems = (gsem0, gsem1)
        idx = ((src_va, dst_va, isema), (src_vb, dst_vb, isemb))

        def idx_copies(b, g, p):
            sv, dv, sem = idx[p]
            return (pltpu.make_async_copy(
                        src_hbm.at[(b * _NT + sid) * ngroups + g], sv, sem),
                    pltpu.make_async_copy(
                        dst_hbm.at[sid * ngroups + g], dv, sem))
        for b_loc in range(b_per_sc):
            b = b_loc * _NSC + cid

            for c in idx_copies(b, 0, 0):
                c.start()
            for c in idx_copies(b, 0, 0):
                c.wait()

            def gather0(jj):
                return pltpu.make_async_copy(
                    x_hbm.at[idx[0][0].at[jj]], rows[jj % _NBUF],
                    gsems[jj % _NBUF])

            gather0(0).start()
            gather0(1).start()

            @pl.when(sid < _NT - 1)
            def _():
                pltpu.sync_copy(z_hbm, acc_sh.at[pl.ds(sid * rows_pt, rows_pt)])

            @pl.when(sid == _NT - 1)
            def _():
                pltpu.sync_copy(
                    z_hbm.at[pl.ds(0, zr_last)],
                    acc_sh.at[pl.ds((_NT - 1) * rows_pt, zr_last)])

            plsc.subcore_barrier()

            @pl.loop(0, ngroups, step=2)
            def _(g):
                for c in idx_copies(b, g + 1, 1):
                    c.start()

                def gather(jj):
                    sv = idx[(jj // _G) % 2][0]
                    return pltpu.make_async_copy(
                        x_hbm.at[sv.at[jj % _G]], rows[jj % _NBUF],
                        gsems[jj % _NBUF])

                for jj in range(2 * _G):
                    if jj == _G - 2:
                        for c in idx_copies(b, g + 1, 1):
                            c.wait()
                    if jj == _G:
                        @pl.when(g + 2 < ngroups)
                        def _():
                            for c in idx_copies(b, g + 2, 0):
                                c.start()
                    gather(jj).wait()
                    dv = idx[(jj // _G) % 2][1]
                    pltpu.sync_copy(rows[jj % _NBUF],
                                    acc_sh.at[dv.at[jj % _G]], add=True)
                    if jj + 2 < 2 * _G:
                        gather(jj + 2).start()
                    elif jj == 2 * _G - 2:
                        @pl.when(g + 2 < ngroups)
                        def _():
                            for c in idx_copies(b, g + 2, 0):
                                c.wait()
                            gather(2 * _G).start()
                    else:
                        @pl.when(g + 2 < ngroups)
                        def _():
                            gather(2 * _G + 1).start()

            plsc.subcore_barrier()

            @pl.when(sid < _NT - 1)
            def _():
                pltpu.sync_copy(
                    acc_sh.at[pl.ds(sid * rows_pt, rows_pt)],
                    out_hbm.at[pl.ds(b * N + sid * rows_pt, rows_pt)],
                )

            @pl.when(sid == _NT - 1)
            def _():
                pltpu.sync_copy(
                    acc_sh.at[pl.ds((_NT - 1) * rows_pt, wr_last)],
                    out_hbm.at[pl.ds(b * N + (_NT - 1) * rows_pt, wr_last)],
                )

            plsc.subcore_barrier()

    kern = pl.kernel(
        body,
        out_type=jax.ShapeDtypeStruct((B * N, D), jnp.float32),
        mesh=mesh,
        scratch_types=[
            pltpu.VMEM((_G, _CH), jnp.int32),
            pltpu.VMEM((_G, _CH), jnp.int32),
            pltpu.VMEM((_G, _CH), jnp.int32),
            pltpu.VMEM((_G, _CH), jnp.int32),
            pltpu.VMEM((_CH, D), jnp.float32),
            pltpu.VMEM((_CH, D), jnp.float32),
            pltpu.SemaphoreType.DMA,
            pltpu.SemaphoreType.DMA,
            pltpu.SemaphoreType.DMA,
            pltpu.SemaphoreType.DMA,
            pltpu.VMEM_SHARED((n_pad, D), jnp.float32),
        ],
    )
    return kern(x2d, srcs, dsts, zrows)


def _combine_body(x_ref, agg_ref, ws_ref, wn_ref, b_ref, o_ref):
    h = jnp.dot(x_ref[...], ws_ref[...], preferred_element_type=jnp.float32)
    h = h + jnp.dot(agg_ref[...], wn_ref[...], preferred_element_type=jnp.float32)
    o_ref[...] = jnp.maximum(h + b_ref[...], 0.0)


def _combine(x2d, agg2d, wsT, wnT, bias):
    M, D = x2d.shape
    BM = 2000
    grid = (M // BM,)
    return pl.pallas_call(
        _combine_body,
        grid=grid,
        in_specs=[
            pl.BlockSpec((BM, D), lambda i: (i, 0)),
            pl.BlockSpec((BM, D), lambda i: (i, 0)),
            pl.BlockSpec((D, D), lambda i: (0, 0)),
            pl.BlockSpec((D, D), lambda i: (0, 0)),
            pl.BlockSpec((1, D), lambda i: (0, 0)),
        ],
        out_specs=pl.BlockSpec((BM, D), lambda i: (i, 0)),
        out_shape=jax.ShapeDtypeStruct((M, D), jnp.float32),
    )(x2d, agg2d, wsT, wnT, bias)


def kernel(x, edge_index, W_self, b_self, W_neigh, b_neigh):
    B, N, D = x.shape
    E = edge_index.shape[0]

    chunk_all = _NT * _G * _CH
    e_pad = ((E + chunk_all - 1) // chunk_all) * chunk_all
    ngroups = e_pad // chunk_all
    src = edge_index[:, 0].astype(jnp.int32)
    dst = edge_index[:, 1].astype(jnp.int32)
    pad = e_pad - E
    if pad:
        src = jnp.concatenate([src, jnp.zeros((pad,), jnp.int32)])
        dst = jnp.concatenate([dst, jnp.full((pad,), N, jnp.int32)])

    n_pad = ((N + 1 + 7) // 8) * 8
    rows_pt = (((n_pad + _NT - 1) // _NT + 7) // 8) * 8

    boffs = (jnp.arange(B, dtype=jnp.int32) * N)[:, None]
    srcs = (src[None, :] + boffs).reshape(B * _NT * ngroups, _G, _CH)
    dsts = dst.reshape(_NT * ngroups, _G, _CH)
    zrows = jnp.zeros((rows_pt, D), jnp.float32)

    x2d = x.reshape(B * N, D)
    agg2d = _sc_scatter_sum(x2d, srcs, dsts, zrows, B, N, D, ngroups, rows_pt, n_pad)

    bias = (b_self + b_neigh).reshape(1, D)
    out2d = _combine(x2d, agg2d, W_self.T, W_neigh.T, bias)
    return out2d.reshape(B, N, D)
